# Initial kernel scaffold; baseline (speedup 1.0000x reference)
import jax
import jax.numpy as jnp
from jax.experimental import pallas as pl


def kernel(X, vertex, edges, degE, degV, alpha, beta, X0, W):
    raise NotImplementedError("write your pallas kernel here")


# speedup vs baseline: 1.9943x; 1.9943x over previous
import functools
import math

import jax
import jax.numpy as jnp
from jax import lax
from jax.experimental import pallas as pl
from jax.experimental.pallas import tpu as pltpu
from jax.experimental.pallas import tpu_sc as plsc

F32 = jnp.float32
I32 = jnp.int32

NW = 32
NCORES = 2
NSUB = 16
B = 128
CW = 16


def _scatter_kernel(n_tab, es, dc, nb, with_counts):
  rz = es // NSUB

  out_type = [jax.ShapeDtypeStruct((NCORES * n_tab * es, dc), F32)]
  scratch = [
      pltpu.VMEM((nb, B), I32),
      pltpu.VMEM((nb, B), I32),
      pltpu.VMEM((B, dc), F32),
      pltpu.VMEM_SHARED((es, dc), F32),
      pltpu.SemaphoreType.DMA,
  ]
  if with_counts:
    out_type.append(jax.ShapeDtypeStruct((NCORES * es, CW), F32))
    scratch.insert(3, pltpu.VMEM((B, CW), F32))
    scratch.insert(5, pltpu.VMEM_SHARED((es, CW), F32))

  mesh = plsc.VectorSubcoreMesh(core_axis_name="c", subcore_axis_name="s")

  @functools.partial(
      pl.kernel, mesh=mesh, out_type=tuple(out_type), scratch_types=scratch,
      compiler_params=pltpu.CompilerParams(use_tc_tiling_on_sc=False))
  def scatter(*refs):
    tabs = refs[:n_tab]
    rest = refs[n_tab:]
    if with_counts:
      (gidx_h, sidx_h, ones_h, zrows_h, zcnt_h, out_s, out_c,
       gidx_v, sidx_v, rows_v, ones_v, acc, cnt, sem) = rest
    else:
      (gidx_h, sidx_h, zrows_h, out_s,
       gidx_v, sidx_v, rows_v, acc, sem) = rest
      cnt = ones_v = out_c = zcnt_h = None

    cid = lax.axis_index("c")
    sid = lax.axis_index("s")
    wid = sid * NCORES + cid

    pltpu.sync_copy(gidx_h.at[wid], gidx_v)
    pltpu.sync_copy(sidx_h.at[wid], sidx_v)
    if with_counts:
      pltpu.sync_copy(ones_h, ones_v)

    for k in range(n_tab):
      pltpu.sync_copy(zrows_h, acc.at[pl.ds(sid * rz, rz)])
      if with_counts and k == 0:
        pltpu.sync_copy(zcnt_h, cnt.at[pl.ds(sid * rz, rz)])
      plsc.subcore_barrier()

      tk = tabs[k]
      do_cnt = with_counts and k == 0

      def body(j, carry):
        pltpu.async_copy(tk.at[gidx_v.at[j]], rows_v, sem).wait()
        pltpu.sync_copy(rows_v, acc.at[sidx_v.at[j]], add=True)
        if do_cnt:
          pltpu.sync_copy(ones_v, cnt.at[sidx_v.at[j]], add=True)
        return carry

      lax.fori_loop(0, nb, body, 0)
      plsc.subcore_barrier()

      base = (cid * n_tab + k) * es + sid * rz
      pltpu.sync_copy(acc.at[pl.ds(sid * rz, rz)], out_s.at[pl.ds(base, rz)])
      if do_cnt:
        cbase = cid * es + sid * rz
        pltpu.sync_copy(cnt.at[pl.ds(sid * rz, rz)],
                        out_c.at[pl.ds(cbase, rz)])

  return scatter


def _edge_scale_body(p_ref, c_ref, de_ref, o_ref):
  p = p_ref[0] + p_ref[1]
  c = c_ref[0, :, 0:1] + c_ref[1, :, 0:1]
  scale = de_ref[...] / jnp.maximum(c, 1.0)
  o_ref[0] = jnp.concatenate(
      [p[k] * scale for k in range(p.shape[0])], axis=1)


def _tail_body(a_ref, b_ref, pv_ref, dv_ref, x0_ref, w_ref, o_ref):
  x = pv_ref[0] + pv_ref[1]
  xv = jnp.concatenate([x[k] for k in range(x.shape[0])], axis=1)
  xv = xv * dv_ref[...]
  g = 0.5 * xv * (1.0 + lax.erf(xv * (1.0 / math.sqrt(2.0))))
  a = a_ref[0, 0]
  b = b_ref[0, 0]
  xi = (1.0 - a) * g + a * x0_ref[...]
  o_ref[...] = (1.0 - b) * xi + b * lax.dot_general(
      xi, w_ref[...], (((1,), (1,)), ((), ())), preferred_element_type=F32)


def kernel(X, vertex, edges, degE, degV, alpha, beta, X0, W):
  N, d = X.shape
  E = degE.shape[0]
  nnz = vertex.shape[0]
  assert d == 256 and N % NSUB == 0

  DCA = 64
  KA = d // DCA
  DCC = 128
  KC = d // DCC
  ES = -(-(E + 1) // 128) * 128
  NS = -(-(N + 1) // 128) * 128
  PAIRS = ((nnz + NW * B - 1) // (NW * B)) * NW * B
  NB = PAIRS // (NW * B)

  Xp = jnp.pad(X, ((0, NS - N), (0, 0)))
  tabsA = tuple(Xp[:, k * DCA:(k + 1) * DCA] for k in range(KA))
  npad = PAIRS - nnz
  vp = jnp.concatenate([vertex, jnp.full((npad,), N, I32)]).reshape(NW, NB, B)
  ep = jnp.concatenate([edges, jnp.full((npad,), E, I32)]).reshape(NW, NB, B)
  ones_src = jnp.ones((B, CW), F32)
  zA = jnp.zeros((ES // NSUB, DCA), F32)
  zAc = jnp.zeros((ES // NSUB, CW), F32)
  zC = jnp.zeros((NS // NSUB, DCC), F32)

  phA = _scatter_kernel(KA, ES, DCA, NB, with_counts=True)
  sums_flat, cnt_flat = phA(*tabsA, vp, ep, ones_src, zA, zAc)
  partial = sums_flat.reshape(NCORES, KA, ES, DCA)
  cnts = cnt_flat.reshape(NCORES, ES, CW)

  degE_p = jnp.pad(degE, (0, ES - E)).reshape(ES, 1)
  RBE = ES // 8
  nsub_a = DCC // DCA
  xe = pl.pallas_call(
      _edge_scale_body,
      grid=(KC, ES // RBE),
      in_specs=[
          pl.BlockSpec((NCORES, nsub_a, RBE, DCA), lambda k, j: (0, k, j, 0)),
          pl.BlockSpec((NCORES, RBE, CW), lambda k, j: (0, j, 0)),
          pl.BlockSpec((RBE, 1), lambda k, j: (j, 0)),
      ],
      out_specs=pl.BlockSpec((1, RBE, DCC), lambda k, j: (k, j, 0)),
      out_shape=jax.ShapeDtypeStruct((KC, ES, DCC), F32),
  )(partial, cnts, degE_p)
  tabsC = tuple(xe[k] for k in range(KC))

  phC = _scatter_kernel(KC, NS, DCC, NB, with_counts=False)
  (pv_flat,) = phC(*tabsC, ep, vp, zC)
  pv = pv_flat.reshape(NCORES, KC, NS, DCC)

  RBN = 400
  a2 = jnp.reshape(jnp.asarray(alpha, F32), (1, 1))
  b2 = jnp.reshape(jnp.asarray(beta, F32), (1, 1))
  degV2 = degV.reshape(N, 1)
  out = pl.pallas_call(
      _tail_body,
      grid=(N // RBN,),
      in_specs=[
          pl.BlockSpec(memory_space=pltpu.SMEM),
          pl.BlockSpec(memory_space=pltpu.SMEM),
          pl.BlockSpec((NCORES, KC, RBN, DCC), lambda j: (0, 0, j, 0)),
          pl.BlockSpec((RBN, 1), lambda j: (j, 0)),
          pl.BlockSpec((RBN, d), lambda j: (j, 0)),
          pl.BlockSpec((d, d), lambda j: (0, 0)),
      ],
      out_specs=pl.BlockSpec((RBN, d), lambda j: (j, 0)),
      out_shape=jax.ShapeDtypeStruct((N, d), F32),
  )(a2, b2, pv, degV2, X0, W)
  return out


# speedup vs baseline: 2.2081x; 1.1072x over previous
import functools
import math

import jax
import jax.numpy as jnp
from jax import lax
from jax.experimental import pallas as pl
from jax.experimental.pallas import tpu as pltpu
from jax.experimental.pallas import tpu_sc as plsc

F32 = jnp.float32
I32 = jnp.int32

NW = 32
NCORES = 2
NSUB = 16
B = 128
CW = 16


def _scatter_kernel(n_tab, es, dc, nb, with_counts):
  rz = es // NSUB

  out_type = [jax.ShapeDtypeStruct((NCORES * n_tab * es, dc), F32)]
  scratch = [
      pltpu.VMEM((nb, B), I32),
      pltpu.VMEM((nb, B), I32),
      pltpu.VMEM((B, dc), F32),
      pltpu.VMEM((B, dc), F32),
      pltpu.VMEM_SHARED((es, dc), F32),
      pltpu.SemaphoreType.DMA,
      pltpu.SemaphoreType.DMA,
      pltpu.SemaphoreType.DMA,
      pltpu.SemaphoreType.DMA,
      pltpu.SemaphoreType.DMA,
  ]
  if with_counts:
    out_type.append(jax.ShapeDtypeStruct((NCORES * es, CW), F32))
    scratch.insert(4, pltpu.VMEM((B, CW), F32))
    scratch.insert(6, pltpu.VMEM_SHARED((es, CW), F32))

  mesh = plsc.VectorSubcoreMesh(core_axis_name="c", subcore_axis_name="s")

  @functools.partial(
      pl.kernel, mesh=mesh, out_type=tuple(out_type), scratch_types=scratch,
      compiler_params=pltpu.CompilerParams(use_tc_tiling_on_sc=False))
  def scatter(*refs):
    tabs = refs[:n_tab]
    rest = refs[n_tab:]
    if with_counts:
      (gidx_h, sidx_h, ones_h, zrows_h, zcnt_h, out_s, out_c,
       gidx_v, sidx_v, rows0_v, rows1_v, ones_v, acc, cnt,
       gsem0, gsem1, ssem0, ssem1, csem) = rest
    else:
      (gidx_h, sidx_h, zrows_h, out_s,
       gidx_v, sidx_v, rows0_v, rows1_v, acc,
       gsem0, gsem1, ssem0, ssem1, csem) = rest
      cnt = ones_v = out_c = zcnt_h = None

    cid = lax.axis_index("c")
    sid = lax.axis_index("s")
    wid = sid * NCORES + cid

    pltpu.sync_copy(gidx_h.at[wid], gidx_v)
    pltpu.sync_copy(sidx_h.at[wid], sidx_v)
    if with_counts:
      pltpu.sync_copy(ones_h, ones_v)

    for k in range(n_tab):
      pltpu.sync_copy(zrows_h, acc.at[pl.ds(sid * rz, rz)])
      if with_counts and k == 0:
        pltpu.sync_copy(zcnt_h, cnt.at[pl.ds(sid * rz, rz)])
      plsc.subcore_barrier()

      tk = tabs[k]
      do_cnt = with_counts and k == 0

      def pair_body(i, refill):
        for (jj, rows, gsem, ssem) in (
            (2 * i, rows0_v, gsem0, ssem0),
            (2 * i + 1, rows1_v, gsem1, ssem1)):
          pltpu.make_async_copy(tk.at[gidx_v.at[jj]], rows, gsem).wait()
          pltpu.async_copy(rows, acc.at[sidx_v.at[jj]], ssem, add=True)
          if do_cnt:
            pltpu.async_copy(ones_v, cnt.at[sidx_v.at[jj]], csem, add=True)
        for (jj, rows, gsem, ssem) in (
            (2 * i, rows0_v, gsem0, ssem0),
            (2 * i + 1, rows1_v, gsem1, ssem1)):
          pltpu.make_async_copy(rows, acc.at[sidx_v.at[jj]], ssem).wait()
          if refill:
            pltpu.async_copy(tk.at[gidx_v.at[jj + 2]], rows, gsem)
          if do_cnt:
            pltpu.make_async_copy(ones_v, cnt.at[sidx_v.at[jj]], csem).wait()

      pltpu.async_copy(tk.at[gidx_v.at[0]], rows0_v, gsem0)
      pltpu.async_copy(tk.at[gidx_v.at[1]], rows1_v, gsem1)
      lax.fori_loop(0, nb // 2 - 1,
                    lambda i, c: (pair_body(i, True), c)[1], 0)
      pair_body(nb // 2 - 1, False)
      plsc.subcore_barrier()

      base = (cid * n_tab + k) * es + sid * rz
      pltpu.sync_copy(acc.at[pl.ds(sid * rz, rz)], out_s.at[pl.ds(base, rz)])
      if do_cnt:
        cbase = cid * es + sid * rz
        pltpu.sync_copy(cnt.at[pl.ds(sid * rz, rz)],
                        out_c.at[pl.ds(cbase, rz)])

  return scatter


def _edge_scale_body(p_ref, c_ref, de_ref, o_ref):
  p = p_ref[0] + p_ref[1]
  c = c_ref[0, :, 0:1] + c_ref[1, :, 0:1]
  scale = de_ref[...] / jnp.maximum(c, 1.0)
  o_ref[0] = jnp.concatenate(
      [p[k] * scale for k in range(p.shape[0])], axis=1)


def _tail_body(a_ref, b_ref, pv_ref, dv_ref, x0_ref, w_ref, o_ref):
  x = pv_ref[0] + pv_ref[1]
  xv = jnp.concatenate([x[k] for k in range(x.shape[0])], axis=1)
  xv = xv * dv_ref[...]
  g = 0.5 * xv * (1.0 + lax.erf(xv * (1.0 / math.sqrt(2.0))))
  a = a_ref[0, 0]
  b = b_ref[0, 0]
  xi = (1.0 - a) * g + a * x0_ref[...]
  o_ref[...] = (1.0 - b) * xi + b * lax.dot_general(
      xi, w_ref[...], (((1,), (1,)), ((), ())), preferred_element_type=F32)


def kernel(X, vertex, edges, degE, degV, alpha, beta, X0, W):
  N, d = X.shape
  E = degE.shape[0]
  nnz = vertex.shape[0]
  assert d == 256 and N % NSUB == 0

  DCA = 64
  KA = d // DCA
  DCC = 128
  KC = d // DCC
  ES = -(-(E + 1) // 128) * 128
  NS = -(-(N + 1) // 128) * 128
  PAIRS = ((nnz + NW * B - 1) // (NW * B)) * NW * B
  NB = PAIRS // (NW * B)

  Xp = jnp.pad(X, ((0, NS - N), (0, 0)))
  tabsA = tuple(Xp[:, k * DCA:(k + 1) * DCA] for k in range(KA))
  npad = PAIRS - nnz
  vp = jnp.concatenate([vertex, jnp.full((npad,), N, I32)]).reshape(NW, NB, B)
  ep = jnp.concatenate([edges, jnp.full((npad,), E, I32)]).reshape(NW, NB, B)
  ones_src = jnp.ones((B, CW), F32)
  zA = jnp.zeros((ES // NSUB, DCA), F32)
  zAc = jnp.zeros((ES // NSUB, CW), F32)
  zC = jnp.zeros((NS // NSUB, DCC), F32)

  phA = _scatter_kernel(KA, ES, DCA, NB, with_counts=True)
  sums_flat, cnt_flat = phA(*tabsA, vp, ep, ones_src, zA, zAc)
  partial = sums_flat.reshape(NCORES, KA, ES, DCA)
  cnts = cnt_flat.reshape(NCORES, ES, CW)

  degE_p = jnp.pad(degE, (0, ES - E)).reshape(ES, 1)
  RBE = ES // 8
  nsub_a = DCC // DCA
  xe = pl.pallas_call(
      _edge_scale_body,
      grid=(KC, ES // RBE),
      in_specs=[
          pl.BlockSpec((NCORES, nsub_a, RBE, DCA), lambda k, j: (0, k, j, 0)),
          pl.BlockSpec((NCORES, RBE, CW), lambda k, j: (0, j, 0)),
          pl.BlockSpec((RBE, 1), lambda k, j: (j, 0)),
      ],
      out_specs=pl.BlockSpec((1, RBE, DCC), lambda k, j: (k, j, 0)),
      out_shape=jax.ShapeDtypeStruct((KC, ES, DCC), F32),
  )(partial, cnts, degE_p)
  tabsC = tuple(xe[k] for k in range(KC))

  phC = _scatter_kernel(KC, NS, DCC, NB, with_counts=False)
  (pv_flat,) = phC(*tabsC, ep, vp, zC)
  pv = pv_flat.reshape(NCORES, KC, NS, DCC)

  RBN = 400
  a2 = jnp.reshape(jnp.asarray(alpha, F32), (1, 1))
  b2 = jnp.reshape(jnp.asarray(beta, F32), (1, 1))
  degV2 = degV.reshape(N, 1)
  out = pl.pallas_call(
      _tail_body,
      grid=(N // RBN,),
      in_specs=[
          pl.BlockSpec(memory_space=pltpu.SMEM),
          pl.BlockSpec(memory_space=pltpu.SMEM),
          pl.BlockSpec((NCORES, KC, RBN, DCC), lambda j: (0, 0, j, 0)),
          pl.BlockSpec((RBN, 1), lambda j: (j, 0)),
          pl.BlockSpec((RBN, d), lambda j: (j, 0)),
          pl.BlockSpec((d, d), lambda j: (0, 0)),
      ],
      out_specs=pl.BlockSpec((RBN, d), lambda j: (j, 0)),
      out_shape=jax.ShapeDtypeStruct((N, d), F32),
  )(a2, b2, pv, degV2, X0, W)
  return out
